# baseline (speedup 1.0000x reference)
import functools

import jax
import jax.numpy as jnp
from jax import lax
from jax.experimental import pallas as pl
from jax.experimental.pallas import tpu as pltpu
from jax.experimental.pallas import tpu_sc as plsc

N = 10000
E = 160000
S = 500
SP = 512
D_IN = 300
D_H = 150
D_OUT = 64
DP1 = 152
DP2 = 64

NC = 2
NS = 16
NW = NC * NS
E_PER_T = 10240
E_PAD = NS * E_PER_T
NROW = N + 16
NN = N

BN = 1000
GRID_N = N // BN


@functools.lru_cache(maxsize=None)
def _make_sc_scatter(d, ch, stages):
    nchunk = E_PER_T // (stages * ch)
    stripe = 640
    last = NN - stripe * (NS - 1)
    mesh = plsc.VectorSubcoreMesh(core_axis_name="c", subcore_axis_name="s",
                                  num_cores=NC, num_subcores=NS)

    @functools.partial(
        pl.kernel,
        out_type=[jax.ShapeDtypeStruct((NN, d), jnp.float32),
                  jax.ShapeDtypeStruct((NN, d), jnp.float32)],
        mesh=mesh,
        scratch_types=[
            pltpu.VMEM((nchunk, ch), jnp.int32),
            pltpu.VMEM((nchunk, ch), jnp.int32),
            pltpu.VMEM((ch, d), jnp.float32),
            pltpu.VMEM((ch, d), jnp.float32),
            pltpu.VMEM_SHARED((NN, d), jnp.float32),
            pltpu.SemaphoreType.DMA,
            pltpu.SemaphoreType.DMA,
        ],
        compiler_params=pltpu.CompilerParams(use_tc_tiling_on_sc=False),
    )
    def sc_scatter(table_a, src_a, dst_a, table_b, src_b, dst_b, zeros_hbm,
                   out_a, out_b,
                   src_v, dst_v, rows0, rows1, acc_sh, sem0, sem1):
        c = lax.axis_index("c")
        s = lax.axis_index("s")

        def run(table_hbm, src_hbm, dst_hbm, out_hbm):
            @pl.when(s < NS - 1)
            def _():
                pltpu.sync_copy(zeros_hbm.at[pl.ds(s * stripe, stripe)],
                                acc_sh.at[pl.ds(s * stripe, stripe)])

            @pl.when(s == NS - 1)
            def _():
                pltpu.sync_copy(zeros_hbm.at[pl.ds(NN - last, last)],
                                acc_sh.at[pl.ds(NN - last, last)])

            plsc.subcore_barrier()

            for stage in range(stages):
                pltpu.sync_copy(src_hbm.at[s, stage], src_v)
                pltpu.sync_copy(dst_hbm.at[s, stage], dst_v)
                pltpu.async_copy(table_hbm.at[src_v.at[0]], rows0, sem0)
                pltpu.async_copy(table_hbm.at[src_v.at[1]], rows1, sem1)

                def body(i, carry):
                    j = 2 * i
                    pltpu.make_async_copy(table_hbm.at[src_v.at[j]], rows0, sem0).wait()
                    pltpu.sync_copy(rows0, acc_sh.at[dst_v.at[j]], add=True)

                    @pl.when(j + 2 < nchunk)
                    def _():
                        pltpu.async_copy(table_hbm.at[src_v.at[j + 2]], rows0, sem0)

                    pltpu.make_async_copy(table_hbm.at[src_v.at[j + 1]], rows1, sem1).wait()
                    pltpu.sync_copy(rows1, acc_sh.at[dst_v.at[j + 1]], add=True)

                    @pl.when(j + 3 < nchunk)
                    def _():
                        pltpu.async_copy(table_hbm.at[src_v.at[j + 3]], rows1, sem1)

                    return carry

                lax.fori_loop(0, nchunk // 2, body, 0)

            plsc.subcore_barrier()

            @pl.when(s < NS - 1)
            def _():
                pltpu.sync_copy(acc_sh.at[pl.ds(s * stripe, stripe)],
                                out_hbm.at[pl.ds(s * stripe, stripe)])

            @pl.when(s == NS - 1)
            def _():
                pltpu.sync_copy(acc_sh.at[pl.ds(NN - last, last)],
                                out_hbm.at[pl.ds(NN - last, last)])

        @pl.when(c == 0)
        def _():
            run(table_a, src_a, dst_a, out_a)

        @pl.when(c == 1)
        def _():
            run(table_b, src_b, dst_b, out_b)

    return sc_scatter


def _sc_l1(*args):
    return _make_sc_scatter(DP1, 64, 2)(*args)


def _sc_l2(*args):
    return _make_sc_scatter(DP2, 128, 1)(*args)


def _prep_edges(edge_index):
    src = edge_index[0]
    dst = edge_index[1]
    pad = E_PAD - E
    src_p = jnp.concatenate([src, jnp.full((pad,), N, jnp.int32)])
    dst_p = jnp.concatenate([dst, jnp.zeros((pad,), jnp.int32)])
    return src_p, dst_p


def _pad_table(t):
    return jnp.concatenate([t, jnp.zeros((NROW - N, t.shape[1]), jnp.float32)], axis=0)


def _tc1_body(x_ref, ws_ref, wn_ref, b1_ref, xs_ref, xnp_ref):
    xb = x_ref[...]
    xs_ref[...] = jnp.dot(xb, ws_ref[...], preferred_element_type=jnp.float32) + b1_ref[...]
    xn = jnp.dot(xb, wn_ref[...], preferred_element_type=jnp.float32)
    col = lax.broadcasted_iota(jnp.int32, (BN, DP1), 1)
    xnp_ref[...] = xn + (col >= D_H).astype(jnp.float32)


def _tc1(x, w_self, w_neigh_pad, b1row):
    return pl.pallas_call(
        _tc1_body,
        grid=(GRID_N,),
        in_specs=[
            pl.BlockSpec((BN, D_IN), lambda i: (i, 0)),
            pl.BlockSpec((D_IN, D_H), lambda i: (0, 0)),
            pl.BlockSpec((D_IN, DP1), lambda i: (0, 0)),
            pl.BlockSpec((1, D_H), lambda i: (0, 0)),
        ],
        out_specs=[
            pl.BlockSpec((BN, D_H), lambda i: (i, 0)),
            pl.BlockSpec((BN, DP1), lambda i: (i, 0)),
        ],
        out_shape=[
            jax.ShapeDtypeStruct((N, D_H), jnp.float32),
            jax.ShapeDtypeStruct((N, DP1), jnp.float32),
        ],
    )(x, w_self, w_neigh_pad, b1row)


def _tc2_body(xs_ref, agg_ref, ws_ref, wn_ref, b2_ref, hs2_ref, hn2_ref, rc_ref):
    a = agg_ref[...]
    cnt = jnp.maximum(a[:, D_H:D_H + 1], 1.0)
    rc = 1.0 / cnt
    mean = a[:, :D_H] * rc
    h1 = jnp.maximum(xs_ref[...] + mean, 0.0)
    hs2_ref[...] = jnp.dot(h1, ws_ref[...], preferred_element_type=jnp.float32) + b2_ref[...]
    hn2_ref[...] = jnp.dot(h1, wn_ref[...], preferred_element_type=jnp.float32)
    rc_ref[...] = jnp.broadcast_to(rc, rc_ref.shape)


def _tc2(xs, agg2, w2s, w2n, b2row):
    return pl.pallas_call(
        _tc2_body,
        grid=(GRID_N,),
        in_specs=[
            pl.BlockSpec((BN, D_H), lambda i: (i, 0)),
            pl.BlockSpec((BN, DP1), lambda i: (i, 0)),
            pl.BlockSpec((D_H, D_OUT), lambda i: (0, 0)),
            pl.BlockSpec((D_H, D_OUT), lambda i: (0, 0)),
            pl.BlockSpec((1, D_OUT), lambda i: (0, 0)),
        ],
        out_specs=[
            pl.BlockSpec((BN, D_OUT), lambda i: (i, 0)),
            pl.BlockSpec((BN, D_OUT), lambda i: (i, 0)),
            pl.BlockSpec((BN, 8), lambda i: (i, 0)),
        ],
        out_shape=[
            jax.ShapeDtypeStruct((N, D_OUT), jnp.float32),
            jax.ShapeDtypeStruct((N, D_OUT), jnp.float32),
            jax.ShapeDtypeStruct((N, 8), jnp.float32),
        ],
    )(xs, agg2, w2s, w2n, b2row)


def _tc3_body(hs2_ref, agg_ref, rc_ref, wg_ref, bg_ref, seg_ref,
              out_ref, h2s, gates, macc, zacc, nacc):
    p = pl.program_id(0)
    i = pl.program_id(1)
    iota_sp = lax.broadcasted_iota(jnp.int32, (BN, SP), 1)
    seg_mat = (seg_ref[...] == iota_sp)

    @pl.when(p == 0)
    def _():
        @pl.when(i == 0)
        def _():
            macc[...] = jnp.full((8, SP), -1e30, jnp.float32)

        h2 = hs2_ref[...] + agg_ref[...] * rc_ref[:, 0:1]
        h2s[pl.ds(i * BN, BN), :] = h2
        g = jnp.sum(h2 * wg_ref[...], axis=1, keepdims=True) + bg_ref[0, 0]
        gates[pl.ds(i * BN, BN), :] = jnp.broadcast_to(g, (BN, 8))
        mg = jnp.where(seg_mat, g, -1e30)
        mb = jnp.max(mg, axis=0, keepdims=True)
        macc[...] = jnp.maximum(macc[...], jnp.broadcast_to(mb, (8, SP)))

    @pl.when(p == 1)
    def _():
        @pl.when(i == 0)
        def _():
            mm = macc[...]
            macc[...] = jnp.where(mm < -1e29, 0.0, mm)
            zacc[...] = jnp.zeros((SP, 8), jnp.float32)
            nacc[...] = jnp.zeros((SP, D_OUT), jnp.float32)

        g = gates[pl.ds(i * BN, BN), 0:1]
        mseg = jnp.max(jnp.where(seg_mat, macc[0:1, :], -3e38), axis=1,
                       keepdims=True)
        e = jnp.exp(g - mseg)
        pf = seg_mat.astype(jnp.float32)
        eh = e * h2s[pl.ds(i * BN, BN), :]
        nacc[...] += lax.dot_general(pf, eh, (((0,), (0,)), ((), ())),
                                     preferred_element_type=jnp.float32)
        zacc[...] += lax.dot_general(pf, jnp.broadcast_to(e, (BN, 8)),
                                     (((0,), (0,)), ((), ())),
                                     preferred_element_type=jnp.float32)

        @pl.when(i == GRID_N - 1)
        def _():
            out_ref[...] = nacc[...] / (zacc[:, 0:1] + 1e-9)


def _tc3(hs2, agg2, rc, wg_row, bg11, seg2d):
    return pl.pallas_call(
        _tc3_body,
        grid=(2, GRID_N),
        in_specs=[
            pl.BlockSpec((BN, D_OUT), lambda p, i: ((1 - p) * i, 0)),
            pl.BlockSpec((BN, DP2), lambda p, i: ((1 - p) * i, 0)),
            pl.BlockSpec((BN, 8), lambda p, i: ((1 - p) * i, 0)),
            pl.BlockSpec((1, D_OUT), lambda p, i: (0, 0)),
            pl.BlockSpec((1, 1), lambda p, i: (0, 0)),
            pl.BlockSpec((BN, 1), lambda p, i: (i, 0)),
        ],
        out_specs=pl.BlockSpec((SP, D_OUT), lambda p, i: (0, 0)),
        out_shape=jax.ShapeDtypeStruct((SP, D_OUT), jnp.float32),
        scratch_shapes=[pltpu.VMEM((N, D_OUT), jnp.float32),
                        pltpu.VMEM((N, 8), jnp.float32),
                        pltpu.VMEM((8, SP), jnp.float32),
                        pltpu.VMEM((SP, 8), jnp.float32),
                        pltpu.VMEM((SP, D_OUT), jnp.float32)],
    )(hs2, agg2, rc, wg_row, bg11, seg2d)


def _tc5_body(sem_ref, syn_ref, hsg_ref, wa_ref, wb_ref, wc_ref, b1_ref,
              w2_ref, b2_ref, out_ref):
    h = (jnp.dot(sem_ref[...], wa_ref[...], preferred_element_type=jnp.float32)
         + jnp.dot(syn_ref[...], wb_ref[...], preferred_element_type=jnp.float32)
         + jnp.dot(hsg_ref[...], wc_ref[...], preferred_element_type=jnp.float32)
         + b1_ref[...])
    h = jnp.maximum(h, 0.0)
    out_ref[...] = jnp.dot(h, w2_ref[...], preferred_element_type=jnp.float32) + b2_ref[...]


def _tc5(sem_feat, syn_feat, hsg_pad, wa, wb, wc, b1row, w2p, b2row):
    return pl.pallas_call(
        _tc5_body,
        out_shape=jax.ShapeDtypeStruct((SP, 8), jnp.float32),
    )(sem_feat, syn_feat, hsg_pad, wa, wb, wc, b1row, w2p, b2row)


def _edge_views(edge_index):
    src_p, dst_p = _prep_edges(edge_index)
    return (src_p.reshape(NS, 2, 80, 64), dst_p.reshape(NS, 2, 80, 64),
            src_p.reshape(NS, 1, 80, 128), dst_p.reshape(NS, 1, 80, 128))


def kernel(syntax_x, syntax_edge_index, syntax_seg, semantic_x, semantic_edge_index,
           semantic_seg, hsg_feature,
           syn_w1s, syn_w1n, syn_b1, syn_w2s, syn_w2n, syn_b2, syn_wg, syn_bg,
           sem_w1s, sem_w1n, sem_b1, sem_w2s, sem_w2n, sem_b2, sem_wg, sem_bg,
           cls_w1, cls_b1, cls_w2, cls_b2):
    sa1, da1, sa2, da2 = _edge_views(syntax_edge_index)
    sb1, db1, sb2, db2 = _edge_views(semantic_edge_index)
    segA = syntax_seg.reshape(N, 1)
    segB = semantic_seg.reshape(N, 1)
    z1 = jnp.zeros((NN, DP1), jnp.float32)
    z2 = jnp.zeros((NN, DP2), jnp.float32)

    xsA, xnpA = _tc1(syntax_x, syn_w1s,
                     jnp.pad(syn_w1n, ((0, 0), (0, DP1 - D_H))), syn_b1.reshape(1, D_H))
    xsB, xnpB = _tc1(semantic_x, sem_w1s,
                     jnp.pad(sem_w1n, ((0, 0), (0, DP1 - D_H))), sem_b1.reshape(1, D_H))

    aggA1, aggB1 = _sc_l1(_pad_table(xnpA), sa1, da1, _pad_table(xnpB), sb1, db1, z1)

    hs2A, hnA, rcA = _tc2(xsA, aggA1, syn_w2s, syn_w2n, syn_b2.reshape(1, D_OUT))
    hs2B, hnB, rcB = _tc2(xsB, aggB1, sem_w2s, sem_w2n, sem_b2.reshape(1, D_OUT))

    aggA2, aggB2 = _sc_l2(_pad_table(hnA), sa2, da2, _pad_table(hnB), sb2, db2, z2)

    syn_feat = _tc3(hs2A, aggA2, rcA, syn_wg.reshape(1, D_OUT),
                    syn_bg.reshape(1, 1), segA)
    sem_feat = _tc3(hs2B, aggB2, rcB, sem_wg.reshape(1, D_OUT),
                    sem_bg.reshape(1, 1), segB)

    hsg_pad = jnp.pad(hsg_feature, ((0, SP - S), (0, 0)))
    wa = cls_w1[:D_OUT]
    wb = cls_w1[D_OUT:2 * D_OUT]
    wc = cls_w1[2 * D_OUT:]
    w2p = jnp.pad(cls_w2, ((0, 0), (0, 8 - 2)))
    b2p = jnp.pad(cls_b2, ((0, 8 - 2),)).reshape(1, 8)

    out = _tc5(sem_feat, syn_feat, hsg_pad, wa, wb, wc,
               cls_b1.reshape(1, 128), w2p, b2p)
    return out[:S, :2]


# speedup vs baseline: 1.1019x; 1.1019x over previous
import functools

import jax
import jax.numpy as jnp
from jax import lax
from jax.experimental import pallas as pl
from jax.experimental.pallas import tpu as pltpu
from jax.experimental.pallas import tpu_sc as plsc

N = 10000
E = 160000
S = 500
SP = 512
D_IN = 300
D_H = 150
D_OUT = 64
DP1 = 152
DP2 = 64

NC = 2
NS = 16
NW = NC * NS
E_PER_T = 10240
E_PAD = NS * E_PER_T
NN = N + 8

BN = 1000
GRID_N = N // BN


@functools.lru_cache(maxsize=None)
def _make_sc_scatter(d, ch, stages):
    nchunk = E_PER_T // (stages * ch)
    stripe = 632
    last = NN - stripe * (NS - 1)
    mesh = plsc.VectorSubcoreMesh(core_axis_name="c", subcore_axis_name="s",
                                  num_cores=NC, num_subcores=NS)

    @functools.partial(
        pl.kernel,
        out_type=[jax.ShapeDtypeStruct((NN, d), jnp.float32),
                  jax.ShapeDtypeStruct((NN, d), jnp.float32)],
        mesh=mesh,
        scratch_types=[
            pltpu.VMEM((nchunk, ch), jnp.int32),
            pltpu.VMEM((nchunk, ch), jnp.int32),
            pltpu.VMEM((ch, d), jnp.float32),
            pltpu.VMEM((ch, d), jnp.float32),
            pltpu.VMEM_SHARED((NN, d), jnp.float32),
            pltpu.SemaphoreType.DMA,
            pltpu.SemaphoreType.DMA,
        ],
        compiler_params=pltpu.CompilerParams(use_tc_tiling_on_sc=False),
    )
    def sc_scatter(table_a, src_a, dst_a, table_b, src_b, dst_b, zeros_hbm,
                   out_a, out_b,
                   src_v, dst_v, rows0, rows1, acc_sh, sem0, sem1):
        c = lax.axis_index("c")
        s = lax.axis_index("s")

        def run(table_hbm, src_hbm, dst_hbm, out_hbm):
            @pl.when(s < NS - 1)
            def _():
                pltpu.sync_copy(zeros_hbm.at[pl.ds(s * stripe, stripe)],
                                acc_sh.at[pl.ds(s * stripe, stripe)])

            @pl.when(s == NS - 1)
            def _():
                pltpu.sync_copy(zeros_hbm.at[pl.ds(NN - last, last)],
                                acc_sh.at[pl.ds(NN - last, last)])

            plsc.subcore_barrier()

            for stage in range(stages):
                pltpu.sync_copy(src_hbm.at[s, stage], src_v)
                pltpu.sync_copy(dst_hbm.at[s, stage], dst_v)
                pltpu.async_copy(table_hbm.at[src_v.at[0]], rows0, sem0)
                pltpu.async_copy(table_hbm.at[src_v.at[1]], rows1, sem1)

                def body(i, carry):
                    j = 2 * i
                    pltpu.make_async_copy(table_hbm.at[src_v.at[j]], rows0, sem0).wait()
                    pltpu.sync_copy(rows0, acc_sh.at[dst_v.at[j]], add=True)

                    @pl.when(j + 2 < nchunk)
                    def _():
                        pltpu.async_copy(table_hbm.at[src_v.at[j + 2]], rows0, sem0)

                    pltpu.make_async_copy(table_hbm.at[src_v.at[j + 1]], rows1, sem1).wait()
                    pltpu.sync_copy(rows1, acc_sh.at[dst_v.at[j + 1]], add=True)

                    @pl.when(j + 3 < nchunk)
                    def _():
                        pltpu.async_copy(table_hbm.at[src_v.at[j + 3]], rows1, sem1)

                    return carry

                lax.fori_loop(0, nchunk // 2, body, 0)

            plsc.subcore_barrier()

            @pl.when(s < NS - 1)
            def _():
                pltpu.sync_copy(acc_sh.at[pl.ds(s * stripe, stripe)],
                                out_hbm.at[pl.ds(s * stripe, stripe)])

            @pl.when(s == NS - 1)
            def _():
                pltpu.sync_copy(acc_sh.at[pl.ds(NN - last, last)],
                                out_hbm.at[pl.ds(NN - last, last)])

        @pl.when(c == 0)
        def _():
            run(table_a, src_a, dst_a, out_a)

        @pl.when(c == 1)
        def _():
            run(table_b, src_b, dst_b, out_b)

    return sc_scatter


def _sc_l1(*args):
    return _make_sc_scatter(DP1, 64, 2)(*args)


def _sc_l2(*args):
    return _make_sc_scatter(DP2, 128, 1)(*args)


def _prep_edges(edge_index):
    src = edge_index[0]
    dst = edge_index[1]
    pad = E_PAD - E
    src_p = jnp.concatenate([src, jnp.zeros((pad,), jnp.int32)])
    dst_p = jnp.concatenate([dst, jnp.full((pad,), N, jnp.int32)])
    return src_p, dst_p


def _tc1_body(x_ref, ws_ref, wn_ref, b1_ref, xs_ref, xnp_ref):
    xb = x_ref[...]
    xs_ref[...] = jnp.dot(xb, ws_ref[...], preferred_element_type=jnp.float32) + b1_ref[...]
    xn = jnp.dot(xb, wn_ref[...], preferred_element_type=jnp.float32)
    col = lax.broadcasted_iota(jnp.int32, (BN, DP1), 1)
    xnp_ref[...] = xn + (col >= D_H).astype(jnp.float32)


def _tc1(x, w_self, w_neigh_pad, b1row):
    return pl.pallas_call(
        _tc1_body,
        grid=(GRID_N,),
        in_specs=[
            pl.BlockSpec((BN, D_IN), lambda i: (i, 0)),
            pl.BlockSpec((D_IN, D_H), lambda i: (0, 0)),
            pl.BlockSpec((D_IN, DP1), lambda i: (0, 0)),
            pl.BlockSpec((1, D_H), lambda i: (0, 0)),
        ],
        out_specs=[
            pl.BlockSpec((BN, D_H), lambda i: (i, 0)),
            pl.BlockSpec((BN, DP1), lambda i: (i, 0)),
        ],
        out_shape=[
            jax.ShapeDtypeStruct((N, D_H), jnp.float32),
            jax.ShapeDtypeStruct((N, DP1), jnp.float32),
        ],
    )(x, w_self, w_neigh_pad, b1row)


def _tc2_body(xs_ref, agg_ref, ws_ref, wn_ref, b2_ref, hs2_ref, hn2_ref, rc_ref):
    a = agg_ref[...]
    cnt = jnp.maximum(a[:, D_H:D_H + 1], 1.0)
    rc = 1.0 / cnt
    mean = a[:, :D_H] * rc
    h1 = jnp.maximum(xs_ref[...] + mean, 0.0)
    hs2_ref[...] = jnp.dot(h1, ws_ref[...], preferred_element_type=jnp.float32) + b2_ref[...]
    hn2_ref[...] = jnp.dot(h1, wn_ref[...], preferred_element_type=jnp.float32)
    rc_ref[...] = jnp.broadcast_to(rc, rc_ref.shape)


def _tc2(xs, agg2, w2s, w2n, b2row):
    return pl.pallas_call(
        _tc2_body,
        grid=(GRID_N,),
        in_specs=[
            pl.BlockSpec((BN, D_H), lambda i: (i, 0)),
            pl.BlockSpec((BN, DP1), lambda i: (i, 0)),
            pl.BlockSpec((D_H, D_OUT), lambda i: (0, 0)),
            pl.BlockSpec((D_H, D_OUT), lambda i: (0, 0)),
            pl.BlockSpec((1, D_OUT), lambda i: (0, 0)),
        ],
        out_specs=[
            pl.BlockSpec((BN, D_OUT), lambda i: (i, 0)),
            pl.BlockSpec((BN, D_OUT), lambda i: (i, 0)),
            pl.BlockSpec((BN, 8), lambda i: (i, 0)),
        ],
        out_shape=[
            jax.ShapeDtypeStruct((N, D_OUT), jnp.float32),
            jax.ShapeDtypeStruct((N, D_OUT), jnp.float32),
            jax.ShapeDtypeStruct((N, 8), jnp.float32),
        ],
    )(xs, agg2, w2s, w2n, b2row)


def _tc3_body(hs2_ref, agg_ref, rc_ref, wg_ref, bg_ref, seg_ref,
              h2_ref, gate_ref, m_ref, macc):
    i = pl.program_id(0)

    @pl.when(i == 0)
    def _():
        macc[...] = jnp.full((8, SP), -1e30, jnp.float32)

    h2 = hs2_ref[...] + agg_ref[...] * rc_ref[:, 0:1]
    h2_ref[...] = h2
    g = jnp.sum(h2 * wg_ref[...], axis=1, keepdims=True) + bg_ref[0, 0]
    gate_ref[...] = jnp.broadcast_to(g, (BN, 8))
    seg_mat = (seg_ref[...] == lax.broadcasted_iota(jnp.int32, (BN, SP), 1))
    mg = jnp.where(seg_mat, g, -1e30)
    mb = jnp.max(mg, axis=0, keepdims=True)
    macc[...] = jnp.maximum(macc[...], jnp.broadcast_to(mb, (8, SP)))

    @pl.when(i == GRID_N - 1)
    def _():
        mm = macc[...]
        m_ref[...] = jnp.where(mm < -1e29, 0.0, mm)


def _tc3(hs2, agg2, rc, wg_row, bg11, seg2d):
    return pl.pallas_call(
        _tc3_body,
        grid=(GRID_N,),
        in_specs=[
            pl.BlockSpec((BN, D_OUT), lambda i: (i, 0)),
            pl.BlockSpec((BN, DP2), lambda i: (i, 0)),
            pl.BlockSpec((BN, 8), lambda i: (i, 0)),
            pl.BlockSpec((1, D_OUT), lambda i: (0, 0)),
            pl.BlockSpec((1, 1), lambda i: (0, 0)),
            pl.BlockSpec((BN, 1), lambda i: (i, 0)),
        ],
        out_specs=[
            pl.BlockSpec((BN, D_OUT), lambda i: (i, 0)),
            pl.BlockSpec((BN, 8), lambda i: (i, 0)),
            pl.BlockSpec((8, SP), lambda i: (0, 0)),
        ],
        out_shape=[
            jax.ShapeDtypeStruct((N, D_OUT), jnp.float32),
            jax.ShapeDtypeStruct((N, 8), jnp.float32),
            jax.ShapeDtypeStruct((8, SP), jnp.float32),
        ],
        scratch_shapes=[pltpu.VMEM((8, SP), jnp.float32)],
    )(hs2, agg2, rc, wg_row, bg11, seg2d)


def _tc4_body(h2_ref, gate_ref, seg_ref, m_ref, out_ref, zacc, nacc):
    i = pl.program_id(0)

    @pl.when(i == 0)
    def _():
        zacc[...] = jnp.zeros((SP, 8), jnp.float32)
        nacc[...] = jnp.zeros((SP, D_OUT), jnp.float32)

    g = gate_ref[:, 0:1]
    seg_mat = (seg_ref[...] == lax.broadcasted_iota(jnp.int32, (BN, SP), 1))
    mseg = jnp.max(jnp.where(seg_mat, m_ref[0:1, :], -3e38), axis=1, keepdims=True)
    e = jnp.exp(g - mseg)
    pf = seg_mat.astype(jnp.float32)
    eh = e * h2_ref[...]
    nacc[...] += lax.dot_general(pf, eh, (((0,), (0,)), ((), ())),
                                 preferred_element_type=jnp.float32)
    zacc[...] += lax.dot_general(pf, jnp.broadcast_to(e, (BN, 8)),
                                 (((0,), (0,)), ((), ())),
                                 preferred_element_type=jnp.float32)

    @pl.when(i == GRID_N - 1)
    def _():
        out_ref[...] = nacc[...] / (zacc[:, 0:1] + 1e-9)


def _tc4(h2, gate, seg2d, m):
    return pl.pallas_call(
        _tc4_body,
        grid=(GRID_N,),
        in_specs=[
            pl.BlockSpec((BN, D_OUT), lambda i: (i, 0)),
            pl.BlockSpec((BN, 8), lambda i: (i, 0)),
            pl.BlockSpec((BN, 1), lambda i: (i, 0)),
            pl.BlockSpec((8, SP), lambda i: (0, 0)),
        ],
        out_specs=pl.BlockSpec((SP, D_OUT), lambda i: (0, 0)),
        out_shape=jax.ShapeDtypeStruct((SP, D_OUT), jnp.float32),
        scratch_shapes=[pltpu.VMEM((SP, 8), jnp.float32),
                        pltpu.VMEM((SP, D_OUT), jnp.float32)],
    )(h2, gate, seg2d, m)


def _tc5_body(sem_ref, syn_ref, hsg_ref, wa_ref, wb_ref, wc_ref, b1_ref,
              w2_ref, b2_ref, out_ref):
    h = (jnp.dot(sem_ref[...], wa_ref[...], preferred_element_type=jnp.float32)
         + jnp.dot(syn_ref[...], wb_ref[...], preferred_element_type=jnp.float32)
         + jnp.dot(hsg_ref[...], wc_ref[...], preferred_element_type=jnp.float32)
         + b1_ref[...])
    h = jnp.maximum(h, 0.0)
    out_ref[...] = jnp.dot(h, w2_ref[...], preferred_element_type=jnp.float32) + b2_ref[...]


def _tc5(sem_feat, syn_feat, hsg_pad, wa, wb, wc, b1row, w2p, b2row):
    return pl.pallas_call(
        _tc5_body,
        out_shape=jax.ShapeDtypeStruct((SP, 8), jnp.float32),
    )(sem_feat, syn_feat, hsg_pad, wa, wb, wc, b1row, w2p, b2row)


def _edge_views(edge_index):
    src_p, dst_p = _prep_edges(edge_index)
    return (src_p.reshape(NS, 2, 80, 64), dst_p.reshape(NS, 2, 80, 64),
            src_p.reshape(NS, 1, 80, 128), dst_p.reshape(NS, 1, 80, 128))


def kernel(syntax_x, syntax_edge_index, syntax_seg, semantic_x, semantic_edge_index,
           semantic_seg, hsg_feature,
           syn_w1s, syn_w1n, syn_b1, syn_w2s, syn_w2n, syn_b2, syn_wg, syn_bg,
           sem_w1s, sem_w1n, sem_b1, sem_w2s, sem_w2n, sem_b2, sem_wg, sem_bg,
           cls_w1, cls_b1, cls_w2, cls_b2):
    sa1, da1, sa2, da2 = _edge_views(syntax_edge_index)
    sb1, db1, sb2, db2 = _edge_views(semantic_edge_index)
    segA = syntax_seg.reshape(N, 1)
    segB = semantic_seg.reshape(N, 1)
    z1 = jnp.zeros((NN, DP1), jnp.float32)
    z2 = jnp.zeros((NN, DP2), jnp.float32)

    xsA, xnpA = _tc1(syntax_x, syn_w1s,
                     jnp.pad(syn_w1n, ((0, 0), (0, DP1 - D_H))), syn_b1.reshape(1, D_H))
    xsB, xnpB = _tc1(semantic_x, sem_w1s,
                     jnp.pad(sem_w1n, ((0, 0), (0, DP1 - D_H))), sem_b1.reshape(1, D_H))

    aggA1, aggB1 = _sc_l1(xnpA, sa1, da1, xnpB, sb1, db1, z1)

    hs2A, hnA, rcA = _tc2(xsA, aggA1, syn_w2s, syn_w2n, syn_b2.reshape(1, D_OUT))
    hs2B, hnB, rcB = _tc2(xsB, aggB1, sem_w2s, sem_w2n, sem_b2.reshape(1, D_OUT))

    aggA2, aggB2 = _sc_l2(hnA, sa2, da2, hnB, sb2, db2, z2)

    h2A, gateA, mA = _tc3(hs2A, aggA2, rcA, syn_wg.reshape(1, D_OUT),
                          syn_bg.reshape(1, 1), segA)
    h2B, gateB, mB = _tc3(hs2B, aggB2, rcB, sem_wg.reshape(1, D_OUT),
                          sem_bg.reshape(1, 1), segB)

    syn_feat = _tc4(h2A, gateA, segA, mA)
    sem_feat = _tc4(h2B, gateB, segB, mB)

    hsg_pad = jnp.pad(hsg_feature, ((0, SP - S), (0, 0)))
    wa = cls_w1[:D_OUT]
    wb = cls_w1[D_OUT:2 * D_OUT]
    wc = cls_w1[2 * D_OUT:]
    w2p = jnp.pad(cls_w2, ((0, 0), (0, 8 - 2)))
    b2p = jnp.pad(cls_b2, ((0, 8 - 2),)).reshape(1, 8)

    out = _tc5(sem_feat, syn_feat, hsg_pad, wa, wb, wc,
               cls_b1.reshape(1, 128), w2p, b2p)
    return out[:S, :2]
